# baseline (speedup 1.0000x reference)
import functools

import jax
import jax.numpy as jnp
from jax import lax
from jax.experimental import pallas as pl
from jax.experimental.pallas import tpu as pltpu
from jax.experimental.pallas import tpu_sc as plsc

N, E, H = 10000, 320000, 128
NC, NS = 2, 16
NW = NC * NS
EPW = E // NW
CH = 80
KCH = EPW // CH
NP_ = 10240
RPT = NP_ // NS
TILE = 2000
NT_E = E // TILE
NT_H = N // TILE
TN = 640
GN = 16
CB = 16000

_mesh = plsc.VectorSubcoreMesh(core_axis_name="c", subcore_axis_name="s")
_f32 = jnp.float32


def _prep1_body(x_ref, eah_ref, w1e_ref, a_ref, b_ref, ch_ref):
    x = x_ref[...]
    w = w1e_ref[...]
    a_ref[...] = jnp.dot(x, w[0:H], preferred_element_type=_f32)
    b_ref[...] = jnp.dot(x, w[H:2 * H], preferred_element_type=_f32)
    ch_ref[...] = jnp.dot(eah_ref[...], w[2 * H:3 * H],
                          preferred_element_type=_f32)


def _prep2_body(gh_ref, ch_ref, d0_ref, d1_ref, u_ref, dh_ref, vh_ref):
    h = gh_ref[...] + ch_ref[...]
    cnt = d0_ref[...][:, 0:1] + d1_ref[...][:, 0:1]
    dinv = lax.rsqrt(1.0 + cnt)
    dh = jnp.broadcast_to(dinv, h.shape)
    u_ref[...] = dh * h
    dh_ref[...] = dh
    vh_ref[...] = dh * dh * h


def _edge_tail_body(ea_ref, g_ref, w1e_ref, w2_ref, b1_ref, b2_ref,
                    eo_ref, ean_ref):
    ea = ea_ref[...]
    c = jnp.dot(ea, w1e_ref[...][2 * H:3 * H], preferred_element_type=_f32)
    he = jnp.maximum(c + g_ref[...] + b1_ref[...], 0.0)
    ean = jnp.dot(he, w2_ref[...], preferred_element_type=_f32) + b2_ref[...]
    ean_ref[...] = ean
    eo_ref[...] = ea + ean


def _edge_head_body(ea_ref, vh_ref, dh_ref, s0_ref, s1_ref,
                    w2_ref, b1_ref, b2_ref, eo_in_ref,
                    eo_ref, ean_ref):
    he = jnp.maximum(
        vh_ref[...] + dh_ref[...] * (s0_ref[...] + s1_ref[...]) + b1_ref[...],
        0.0)
    ean = jnp.dot(he, w2_ref[...], preferred_element_type=_f32) + b2_ref[...]
    ean_ref[...] = ean
    eo_ref[...] = ea_ref[...] + ean


def _combine_body(p_ref, q_ref, o_ref):
    o_ref[...] = p_ref[...] + q_ref[...]


def _prep_node_body(x_ref, a0_ref, a1_ref, a2_ref, a3_ref, dh_ref, w1n_ref,
                    un_ref, vn_ref):
    w = w1n_ref[...]
    agg = a0_ref[...] + a1_ref[...] + a2_ref[...] + a3_ref[...]
    hx = (jnp.dot(x_ref[...], w[0:H], preferred_element_type=_f32)
          + jnp.dot(agg, w[H:2 * H], preferred_element_type=_f32))
    dh = dh_ref[...]
    un_ref[...] = dh * hx
    vn_ref[...] = dh * dh * hx


def _final_body(x_ref, vn_ref, dh_ref, s0_ref, s1_ref, w2n_ref, b1_ref,
                b2_ref, xo_ref):
    hn = jnp.maximum(
        vn_ref[...] + dh_ref[...] * (s0_ref[...] + s1_ref[...]) + b1_ref[...],
        0.0)
    xo_ref[...] = (x_ref[...]
                   + jnp.dot(hn, w2n_ref[...], preferred_element_type=_f32)
                   + b2_ref[...])


def _row_spec(nt):
    return pl.BlockSpec((TILE, H), lambda i: (i, 0))


def _n_spec():
    return pl.BlockSpec((TN, H), lambda i: (i, 0))


def _n2_spec():
    return pl.BlockSpec((TN, H), lambda i: (GN + i, 0))


def _head_spec():
    return pl.BlockSpec((TILE, H), lambda i: (jnp.minimum(i, NT_H - 1), 0))


def _full_spec(shape):
    return pl.BlockSpec(shape, lambda i: tuple(0 for _ in shape))


CHG = 200
KCHG = EPW // CHG


def _sc_build_g_body(a_hbm, b_hbm, src1_hbm, dst1_hbm, g_out,
                     idxs_v, idxd_v, a0_v, b0_v, a1_v, b1_v,
                     sa0, sb0, sa1, sb1):
    cid = lax.axis_index("c")
    sid = lax.axis_index("s")
    wid = sid * NC + cid
    pltpu.sync_copy(src1_hbm.at[wid], idxs_v)
    pltpu.sync_copy(dst1_hbm.at[wid], idxd_v)

    def slc(k):
        return pl.ds(k * CHG, CHG)

    def add_and_flush(abuf, bbuf, k):
        def row_add(r, c2):
            for lane in range(H // 16):
                sl = pl.ds(lane * 16, 16)
                abuf[r, sl] = abuf[r, sl] + bbuf[r, sl]
            return c2

        lax.fori_loop(0, CHG, row_add, 0)
        pltpu.sync_copy(abuf, g_out.at[pl.ds(wid * EPW + k * CHG, CHG)])

    def wait(buf, sem):
        pltpu.make_async_copy(a_hbm.at[pl.ds(0, CHG)], buf, sem).wait()

    pltpu.async_copy(a_hbm.at[idxs_v.at[slc(0)]], a0_v, sa0)
    pltpu.async_copy(b_hbm.at[idxd_v.at[slc(0)]], b0_v, sb0)

    def pair(p, carry):
        k0 = 2 * p
        k1 = k0 + 1
        pltpu.async_copy(a_hbm.at[idxs_v.at[slc(k1)]], a1_v, sa1)
        pltpu.async_copy(b_hbm.at[idxd_v.at[slc(k1)]], b1_v, sb1)
        wait(a0_v, sa0)
        wait(b0_v, sb0)
        add_and_flush(a0_v, b0_v, k0)

        @pl.when(p < KCHG // 2 - 1)
        def _():
            pltpu.async_copy(a_hbm.at[idxs_v.at[slc(k0 + 2)]], a0_v, sa0)
            pltpu.async_copy(b_hbm.at[idxd_v.at[slc(k0 + 2)]], b0_v, sb0)

        wait(a1_v, sa1)
        wait(b1_v, sb1)
        add_and_flush(a1_v, b1_v, k1)
        return carry

    lax.fori_loop(0, KCHG // 2, pair, 0)


_sc_build_g = pl.kernel(
    _sc_build_g_body,
    out_type=jax.ShapeDtypeStruct((E, H), _f32),
    mesh=_mesh,
    scratch_types=[
        pltpu.VMEM((EPW,), jnp.int32),
        pltpu.VMEM((EPW,), jnp.int32),
        pltpu.VMEM((CHG, H), _f32),
        pltpu.VMEM((CHG, H), _f32),
        pltpu.VMEM((CHG, H), _f32),
        pltpu.VMEM((CHG, H), _f32),
        pltpu.SemaphoreType.DMA,
        pltpu.SemaphoreType.DMA,
        pltpu.SemaphoreType.DMA,
        pltpu.SemaphoreType.DMA,
    ],
)


def _sc_deg_body(dst2_hbm, deg_out, idxd_v, ones_v, dacc_sh):
    cid = lax.axis_index("c")
    sid = lax.axis_index("s")
    wid = sid * NC + cid
    pltpu.sync_copy(dst2_hbm.at[wid], idxd_v)

    def zrow(r, c2):
        for lane in range(H // 16):
            ones_v[r, pl.ds(lane * 16, 16)] = jnp.zeros((16,), _f32)
        return c2

    lax.fori_loop(0, CH, zrow, 0)

    def zcopy(j, c2):
        pltpu.sync_copy(ones_v, dacc_sh.at[pl.ds(sid * RPT + j * CH, CH)])
        return c2

    lax.fori_loop(0, RPT // CH, zcopy, 0)

    def fill_row(r, c2):
        for lane in range(H // 16):
            ones_v[r, pl.ds(lane * 16, 16)] = jnp.ones((16,), _f32)
        return c2

    lax.fori_loop(0, CH, fill_row, 0)
    plsc.subcore_barrier()

    def chunk(k, carry):
        pltpu.sync_copy(ones_v, dacc_sh.at[idxd_v.at[k]], add=True)
        return carry

    lax.fori_loop(0, KCH, chunk, 0)
    plsc.subcore_barrier()

    def dump(j, c2):
        pltpu.sync_copy(dacc_sh.at[pl.ds(sid * RPT + j * CH, CH)], ones_v)
        pltpu.sync_copy(ones_v,
                        deg_out.at[pl.ds(cid * NP_ + sid * RPT + j * CH, CH)])
        return c2

    lax.fori_loop(0, RPT // CH, dump, 0)


_sc_deg = pl.kernel(
    _sc_deg_body,
    out_type=jax.ShapeDtypeStruct((NC * NP_, H), _f32),
    mesh=_mesh,
    scratch_types=[
        pltpu.VMEM((KCH, CH), jnp.int32),
        pltpu.VMEM((CH, H), _f32),
        pltpu.VMEM_SHARED((NP_, H), _f32),
    ],
)


def _sc_gs_body(tbl_hbm, src1_hbm, dst2_hbm, out_part,
                idxs_v, idxd_v, buf0_v, buf1_v, acc_sh, sem0, sem1):
    cid = lax.axis_index("c")
    sid = lax.axis_index("s")
    wid = sid * NC + cid
    pltpu.sync_copy(src1_hbm.at[wid], idxs_v)
    pltpu.sync_copy(dst2_hbm.at[wid], idxd_v)

    def zrow(r, c2):
        for lane in range(H // 16):
            buf0_v[r, pl.ds(lane * 16, 16)] = jnp.zeros((16,), _f32)
        return c2

    lax.fori_loop(0, CH, zrow, 0)

    def zcopy(j, c2):
        pltpu.sync_copy(buf0_v, acc_sh.at[pl.ds(sid * RPT + j * CH, CH)])
        return c2

    lax.fori_loop(0, RPT // CH, zcopy, 0)
    plsc.subcore_barrier()

    def slc(k):
        return pl.ds(k * CH, CH)

    def wait(buf, sem):
        pltpu.make_async_copy(tbl_hbm.at[pl.ds(0, CH)], buf, sem).wait()

    pltpu.async_copy(tbl_hbm.at[idxs_v.at[slc(0)]], buf0_v, sem0)

    def pair(p, carry):
        k0 = 2 * p
        k1 = k0 + 1
        pltpu.async_copy(tbl_hbm.at[idxs_v.at[slc(k1)]], buf1_v, sem1)
        wait(buf0_v, sem0)
        pltpu.sync_copy(buf0_v, acc_sh.at[idxd_v.at[k0]], add=True)
        pltpu.async_copy(tbl_hbm.at[idxs_v.at[slc(k0 + 2)]], buf0_v, sem0)
        wait(buf1_v, sem1)
        pltpu.sync_copy(buf1_v, acc_sh.at[idxd_v.at[k1]], add=True)
        return carry

    lax.fori_loop(0, KCH // 2, pair, 0)
    wait(buf0_v, sem0)
    pltpu.sync_copy(buf0_v, acc_sh.at[idxd_v.at[KCH - 1]], add=True)
    plsc.subcore_barrier()

    def dump(j, c2):
        pltpu.sync_copy(acc_sh.at[pl.ds(sid * RPT + j * CH, CH)], buf0_v)
        pltpu.sync_copy(buf0_v,
                        out_part.at[pl.ds(cid * CB + sid * RPT + j * CH, CH)])
        return c2

    lax.fori_loop(0, RPT // CH, dump, 0)


_sc_gather_scatter = pl.kernel(
    _sc_gs_body,
    out_type=jax.ShapeDtypeStruct((CB + NP_, H), _f32),
    mesh=_mesh,
    scratch_types=[
        pltpu.VMEM((EPW,), jnp.int32),
        pltpu.VMEM((KCH, CH), jnp.int32),
        pltpu.VMEM((CH, H), _f32),
        pltpu.VMEM((CH, H), _f32),
        pltpu.VMEM_SHARED((NP_, H), _f32),
        pltpu.SemaphoreType.DMA,
        pltpu.SemaphoreType.DMA,
    ],
)


def _sc_agg_body(ean_hbm, dst2_hbm, out_part,
                 idxd_v, buf0_v, buf1_v, acc_sh, sem0, sem1):
    cid = lax.axis_index("c")
    sid = lax.axis_index("s")
    wid = sid * NC + cid
    pltpu.sync_copy(dst2_hbm.at[wid], idxd_v)

    def zrow(r, c2):
        for lane in range(H // 16):
            buf0_v[r, pl.ds(lane * 16, 16)] = jnp.zeros((16,), _f32)
        return c2

    lax.fori_loop(0, CH, zrow, 0)

    def zcopy(j, c2):
        pltpu.sync_copy(buf0_v, acc_sh.at[pl.ds(sid * RPT + j * CH, CH)])
        return c2

    lax.fori_loop(0, RPT // CH, zcopy, 0)
    plsc.subcore_barrier()

    def ld(k, buf, sem):
        pltpu.async_copy(ean_hbm.at[pl.ds(wid * EPW + k * CH, CH)], buf, sem)

    def wait(buf, sem):
        pltpu.make_async_copy(ean_hbm.at[pl.ds(0, CH)], buf, sem).wait()

    @pl.when(wid > 0)
    def _():
        ld(0, buf0_v, sem0)

        def pair(p, carry):
            k0 = 2 * p
            k1 = k0 + 1
            ld(k1, buf1_v, sem1)
            wait(buf0_v, sem0)
            pltpu.sync_copy(buf0_v, acc_sh.at[idxd_v.at[k0]], add=True)
            ld(k0 + 2, buf0_v, sem0)
            wait(buf1_v, sem1)
            pltpu.sync_copy(buf1_v, acc_sh.at[idxd_v.at[k1]], add=True)
            return carry

        lax.fori_loop(0, KCH // 2, pair, 0)
        wait(buf0_v, sem0)
        pltpu.sync_copy(buf0_v, acc_sh.at[idxd_v.at[KCH - 1]], add=True)

    plsc.subcore_barrier()

    def dump(j, c2):
        pltpu.sync_copy(acc_sh.at[pl.ds(sid * RPT + j * CH, CH)], buf0_v)
        pltpu.sync_copy(buf0_v,
                        out_part.at[pl.ds(cid * NP_ + sid * RPT + j * CH, CH)])
        return c2

    lax.fori_loop(0, RPT // CH, dump, 0)


_sc_agg = pl.kernel(
    _sc_agg_body,
    out_type=jax.ShapeDtypeStruct((NC * NP_, H), _f32),
    mesh=_mesh,
    scratch_types=[
        pltpu.VMEM((KCH, CH), jnp.int32),
        pltpu.VMEM((CH, H), _f32),
        pltpu.VMEM((CH, H), _f32),
        pltpu.VMEM_SHARED((NP_, H), _f32),
        pltpu.SemaphoreType.DMA,
        pltpu.SemaphoreType.DMA,
    ],
)


def _sc_agg_head_body(eanh_hbm, dst2_hbm, out_part,
                      idxd_v, buf_v, acc_sh, sem):
    cid = lax.axis_index("c")
    sid = lax.axis_index("s")
    wid = sid * NC + cid
    pltpu.sync_copy(dst2_hbm.at[0], idxd_v)

    def zrow(r, c2):
        for lane in range(H // 16):
            buf_v[r, pl.ds(lane * 16, 16)] = jnp.zeros((16,), _f32)
        return c2

    lax.fori_loop(0, CH, zrow, 0)

    def zcopy(j, c2):
        pltpu.sync_copy(buf_v, acc_sh.at[pl.ds(sid * RPT + j * CH, CH)])
        return c2

    lax.fori_loop(0, RPT // CH, zcopy, 0)
    plsc.subcore_barrier()

    for j in range(4):
        c = wid * 4 + j

        @pl.when(c < KCH)
        def _():
            pltpu.async_copy(eanh_hbm.at[pl.ds(c * CH, CH)], buf_v, sem).wait()
            pltpu.sync_copy(buf_v, acc_sh.at[idxd_v.at[c]], add=True)

    plsc.subcore_barrier()

    def dump(j, c2):
        pltpu.sync_copy(acc_sh.at[pl.ds(sid * RPT + j * CH, CH)], buf_v)
        pltpu.sync_copy(buf_v,
                        out_part.at[pl.ds(cid * NP_ + sid * RPT + j * CH, CH)])
        return c2

    lax.fori_loop(0, RPT // CH, dump, 0)


_sc_agg_head = pl.kernel(
    _sc_agg_head_body,
    out_type=jax.ShapeDtypeStruct((NC * NP_, H), _f32),
    mesh=_mesh,
    scratch_types=[
        pltpu.VMEM((KCH, CH), jnp.int32),
        pltpu.VMEM((CH, H), _f32),
        pltpu.VMEM_SHARED((NP_, H), _f32),
        pltpu.SemaphoreType.DMA,
    ],
)


def kernel(x, edge_attr, edge_index, W1e, b1e, W2e, b2e, W1n, b1n, W2n, b2n):
    src1 = edge_index[0].reshape(NW, EPW)
    dst1 = edge_index[1].reshape(NW, EPW)
    dst2 = edge_index[1].reshape(NW, KCH, CH)
    b1e2 = b1e.reshape(1, H)
    b2e2 = b2e.reshape(1, H)
    b1n2 = b1n.reshape(1, H)
    b2n2 = b2n.reshape(1, H)

    a_mat, b_mat, c_head = pl.pallas_call(
        _prep1_body,
        grid=(GN,),
        in_specs=[_n_spec(), _n_spec(), _full_spec((3 * H, H))],
        out_specs=[_n_spec()] * 3,
        out_shape=[jax.ShapeDtypeStruct((N, H), _f32)] * 3,
    )(x, edge_attr, W1e)

    degf = _sc_deg(dst2)
    g_mat = _sc_build_g(a_mat, b_mat, src1, dst1)

    u_mat, dinvh, v_head = pl.pallas_call(
        _prep2_body,
        grid=(GN,),
        in_specs=[_n_spec(), _n_spec(), _n_spec(), _n2_spec()],
        out_specs=[_n_spec()] * 3,
        out_shape=[jax.ShapeDtypeStruct((N, H), _f32)] * 3,
    )(g_mat, c_head, degf, degf)

    sepf = _sc_gather_scatter(u_mat, src1, dst2)

    _tail_spec = pl.BlockSpec((TILE, H), lambda i: (NT_H + i, 0))
    eo_t, ean_t = pl.pallas_call(
        _edge_tail_body,
        grid=(NT_E - NT_H,),
        in_specs=[_tail_spec, _tail_spec,
                  _full_spec((3 * H, H)), _full_spec((H, H)),
                  _full_spec((1, H)), _full_spec((1, H))],
        out_specs=[_tail_spec] * 2,
        out_shape=[jax.ShapeDtypeStruct((E, H), _f32)] * 2,
    )(edge_attr, g_mat, W1e, W2e, b1e2, b2e2)

    aggf_t = _sc_agg(ean_t, dst2)

    _hrow = pl.BlockSpec((TILE, H), lambda i: (i, 0))
    edge_out, ean_h = pl.pallas_call(
        _edge_head_body,
        grid=(NT_H,),
        in_specs=[_hrow, _hrow, _hrow, _hrow,
                  pl.BlockSpec((TILE, H), lambda i: (CB // TILE + i, 0)),
                  _full_spec((H, H)), _full_spec((1, H)), _full_spec((1, H)),
                  _hrow],
        out_specs=[_hrow, _hrow],
        out_shape=[jax.ShapeDtypeStruct((E, H), _f32),
                   jax.ShapeDtypeStruct((N, H), _f32)],
        input_output_aliases={8: 0},
    )(edge_attr, v_head, dinvh, sepf, sepf, W2e, b1e2, b2e2, eo_t)

    aggf_h = _sc_agg_head(ean_h, dst2)

    u_n, v_n = pl.pallas_call(
        _prep_node_body,
        grid=(GN,),
        in_specs=[_n_spec(), _n_spec(), _n2_spec(), _n_spec(), _n2_spec(),
                  _n_spec(), _full_spec((2 * H, H))],
        out_specs=[_n_spec()] * 2,
        out_shape=[jax.ShapeDtypeStruct((N, H), _f32)] * 2,
    )(x, aggf_t, aggf_t, aggf_h, aggf_h, dinvh, W1n)

    snf = _sc_gather_scatter(u_n, src1, dst2)

    x_out = pl.pallas_call(
        _final_body,
        grid=(GN,),
        in_specs=[_n_spec()] * 3
        + [_n_spec(), pl.BlockSpec((TN, H), lambda i: (CB // TN + i, 0))]
        + [_full_spec((H, H)), _full_spec((1, H)), _full_spec((1, H))],
        out_specs=_n_spec(),
        out_shape=jax.ShapeDtypeStruct((N, H), _f32),
    )(x, v_n, dinvh, snf, snf, W2n, b1n2, b2n2)

    return (x_out, edge_out)


# speedup vs baseline: 1.0380x; 1.0380x over previous
import functools

import jax
import jax.numpy as jnp
from jax import lax
from jax.experimental import pallas as pl
from jax.experimental.pallas import tpu as pltpu
from jax.experimental.pallas import tpu_sc as plsc

N, E, H = 10000, 320000, 128
NC, NS = 2, 16
NW = NC * NS
EPW = E // NW
CH = 80
KCH = EPW // CH
NP_ = 10240
RPT = NP_ // NS
TILE = 2000
NT_E = E // TILE
NT_H = N // TILE
TN = 640
GN = 16
CB = 16000

_mesh = plsc.VectorSubcoreMesh(core_axis_name="c", subcore_axis_name="s")
_f32 = jnp.float32


def _prep1_body(x_ref, eah_ref, w1e_ref, a_ref, b_ref, ch_ref):
    x = x_ref[...]
    w = w1e_ref[...]
    a_ref[...] = jnp.dot(x, w[0:H], preferred_element_type=_f32)
    b_ref[...] = jnp.dot(x, w[H:2 * H], preferred_element_type=_f32)
    ch_ref[...] = jnp.dot(eah_ref[...], w[2 * H:3 * H],
                          preferred_element_type=_f32)


def _prep2_body(gh_ref, ch_ref, d0_ref, d1_ref, u_ref, dh_ref, vh_ref):
    h = gh_ref[...] + ch_ref[...]
    cnt = d0_ref[...][:, 0:1] + d1_ref[...][:, 0:1]
    dinv = lax.rsqrt(1.0 + cnt)
    dh = jnp.broadcast_to(dinv, h.shape)
    u_ref[...] = dh * h
    dh_ref[...] = dh
    vh_ref[...] = dh * dh * h


def _edge_tail_body(ea_ref, g_ref, w1e_ref, w2_ref, b1_ref, b2_ref,
                    eo_ref, ean_ref):
    ea = ea_ref[...]
    c = jnp.dot(ea, w1e_ref[...][2 * H:3 * H], preferred_element_type=_f32)
    he = jnp.maximum(c + g_ref[...] + b1_ref[...], 0.0)
    ean = jnp.dot(he, w2_ref[...], preferred_element_type=_f32) + b2_ref[...]
    ean_ref[...] = ean
    eo_ref[...] = ea + ean


def _edge_head_body(ea_ref, vh_ref, dh_ref, s0_ref, s1_ref,
                    w2_ref, b1_ref, b2_ref, eo_in_ref, ean_in_ref,
                    eo_ref, ean_ref):
    he = jnp.maximum(
        vh_ref[...] + dh_ref[...] * (s0_ref[...] + s1_ref[...]) + b1_ref[...],
        0.0)
    ean = jnp.dot(he, w2_ref[...], preferred_element_type=_f32) + b2_ref[...]
    ean_ref[...] = ean
    eo_ref[...] = ea_ref[...] + ean


def _combine_body(p_ref, q_ref, o_ref):
    o_ref[...] = p_ref[...] + q_ref[...]


def _prep_node_body(x_ref, a0_ref, a1_ref, dh_ref, w1n_ref, un_ref, vn_ref):
    w = w1n_ref[...]
    hx = (jnp.dot(x_ref[...], w[0:H], preferred_element_type=_f32)
          + jnp.dot(a0_ref[...] + a1_ref[...], w[H:2 * H],
                    preferred_element_type=_f32))
    dh = dh_ref[...]
    un_ref[...] = dh * hx
    vn_ref[...] = dh * dh * hx


def _final_body(x_ref, vn_ref, dh_ref, s0_ref, s1_ref, w2n_ref, b1_ref,
                b2_ref, xo_ref):
    hn = jnp.maximum(
        vn_ref[...] + dh_ref[...] * (s0_ref[...] + s1_ref[...]) + b1_ref[...],
        0.0)
    xo_ref[...] = (x_ref[...]
                   + jnp.dot(hn, w2n_ref[...], preferred_element_type=_f32)
                   + b2_ref[...])


def _row_spec(nt):
    return pl.BlockSpec((TILE, H), lambda i: (i, 0))


def _n_spec():
    return pl.BlockSpec((TN, H), lambda i: (i, 0))


def _n2_spec():
    return pl.BlockSpec((TN, H), lambda i: (GN + i, 0))


def _head_spec():
    return pl.BlockSpec((TILE, H), lambda i: (jnp.minimum(i, NT_H - 1), 0))


def _full_spec(shape):
    return pl.BlockSpec(shape, lambda i: tuple(0 for _ in shape))


CHG = 200
KCHG = EPW // CHG


def _sc_build_g_body(a_hbm, b_hbm, src1_hbm, dst1_hbm, g_out,
                     idxs_v, idxd_v, a0_v, b0_v, a1_v, b1_v,
                     sa0, sb0, sa1, sb1):
    cid = lax.axis_index("c")
    sid = lax.axis_index("s")
    wid = sid * NC + cid
    pltpu.sync_copy(src1_hbm.at[wid], idxs_v)
    pltpu.sync_copy(dst1_hbm.at[wid], idxd_v)

    def slc(k):
        return pl.ds(k * CHG, CHG)

    def add_and_flush(abuf, bbuf, k):
        def row_add(r, c2):
            for lane in range(H // 16):
                sl = pl.ds(lane * 16, 16)
                abuf[r, sl] = abuf[r, sl] + bbuf[r, sl]
            return c2

        lax.fori_loop(0, CHG, row_add, 0)
        pltpu.sync_copy(abuf, g_out.at[pl.ds(wid * EPW + k * CHG, CHG)])

    def wait(buf, sem):
        pltpu.make_async_copy(a_hbm.at[pl.ds(0, CHG)], buf, sem).wait()

    pltpu.async_copy(a_hbm.at[idxs_v.at[slc(0)]], a0_v, sa0)
    pltpu.async_copy(b_hbm.at[idxd_v.at[slc(0)]], b0_v, sb0)

    def pair(p, carry):
        k0 = 2 * p
        k1 = k0 + 1
        pltpu.async_copy(a_hbm.at[idxs_v.at[slc(k1)]], a1_v, sa1)
        pltpu.async_copy(b_hbm.at[idxd_v.at[slc(k1)]], b1_v, sb1)
        wait(a0_v, sa0)
        wait(b0_v, sb0)
        add_and_flush(a0_v, b0_v, k0)

        @pl.when(p < KCHG // 2 - 1)
        def _():
            pltpu.async_copy(a_hbm.at[idxs_v.at[slc(k0 + 2)]], a0_v, sa0)
            pltpu.async_copy(b_hbm.at[idxd_v.at[slc(k0 + 2)]], b0_v, sb0)

        wait(a1_v, sa1)
        wait(b1_v, sb1)
        add_and_flush(a1_v, b1_v, k1)
        return carry

    lax.fori_loop(0, KCHG // 2, pair, 0)


_sc_build_g = pl.kernel(
    _sc_build_g_body,
    out_type=jax.ShapeDtypeStruct((E, H), _f32),
    mesh=_mesh,
    scratch_types=[
        pltpu.VMEM((EPW,), jnp.int32),
        pltpu.VMEM((EPW,), jnp.int32),
        pltpu.VMEM((CHG, H), _f32),
        pltpu.VMEM((CHG, H), _f32),
        pltpu.VMEM((CHG, H), _f32),
        pltpu.VMEM((CHG, H), _f32),
        pltpu.SemaphoreType.DMA,
        pltpu.SemaphoreType.DMA,
        pltpu.SemaphoreType.DMA,
        pltpu.SemaphoreType.DMA,
    ],
)


def _sc_deg_body(dst2_hbm, deg_out, idxd_v, ones_v, dacc_sh):
    cid = lax.axis_index("c")
    sid = lax.axis_index("s")
    wid = sid * NC + cid
    pltpu.sync_copy(dst2_hbm.at[wid], idxd_v)

    def zrow(r, c2):
        for lane in range(H // 16):
            ones_v[r, pl.ds(lane * 16, 16)] = jnp.zeros((16,), _f32)
        return c2

    lax.fori_loop(0, CH, zrow, 0)

    def zcopy(j, c2):
        pltpu.sync_copy(ones_v, dacc_sh.at[pl.ds(sid * RPT + j * CH, CH)])
        return c2

    lax.fori_loop(0, RPT // CH, zcopy, 0)

    def fill_row(r, c2):
        for lane in range(H // 16):
            ones_v[r, pl.ds(lane * 16, 16)] = jnp.ones((16,), _f32)
        return c2

    lax.fori_loop(0, CH, fill_row, 0)
    plsc.subcore_barrier()

    def chunk(k, carry):
        pltpu.sync_copy(ones_v, dacc_sh.at[idxd_v.at[k]], add=True)
        return carry

    lax.fori_loop(0, KCH, chunk, 0)
    plsc.subcore_barrier()

    def dump(j, c2):
        pltpu.sync_copy(dacc_sh.at[pl.ds(sid * RPT + j * CH, CH)], ones_v)
        pltpu.sync_copy(ones_v,
                        deg_out.at[pl.ds(cid * NP_ + sid * RPT + j * CH, CH)])
        return c2

    lax.fori_loop(0, RPT // CH, dump, 0)


_sc_deg = pl.kernel(
    _sc_deg_body,
    out_type=jax.ShapeDtypeStruct((NC * NP_, H), _f32),
    mesh=_mesh,
    scratch_types=[
        pltpu.VMEM((KCH, CH), jnp.int32),
        pltpu.VMEM((CH, H), _f32),
        pltpu.VMEM_SHARED((NP_, H), _f32),
    ],
)


def _sc_gs_body(tbl_hbm, src1_hbm, dst2_hbm, out_part,
                idxs_v, idxd_v, buf0_v, buf1_v, acc_sh, sem0, sem1):
    cid = lax.axis_index("c")
    sid = lax.axis_index("s")
    wid = sid * NC + cid
    pltpu.sync_copy(src1_hbm.at[wid], idxs_v)
    pltpu.sync_copy(dst2_hbm.at[wid], idxd_v)

    def zrow(r, c2):
        for lane in range(H // 16):
            buf0_v[r, pl.ds(lane * 16, 16)] = jnp.zeros((16,), _f32)
        return c2

    lax.fori_loop(0, CH, zrow, 0)

    def zcopy(j, c2):
        pltpu.sync_copy(buf0_v, acc_sh.at[pl.ds(sid * RPT + j * CH, CH)])
        return c2

    lax.fori_loop(0, RPT // CH, zcopy, 0)
    plsc.subcore_barrier()

    def slc(k):
        return pl.ds(k * CH, CH)

    def wait(buf, sem):
        pltpu.make_async_copy(tbl_hbm.at[pl.ds(0, CH)], buf, sem).wait()

    pltpu.async_copy(tbl_hbm.at[idxs_v.at[slc(0)]], buf0_v, sem0)

    def pair(p, carry):
        k0 = 2 * p
        k1 = k0 + 1
        pltpu.async_copy(tbl_hbm.at[idxs_v.at[slc(k1)]], buf1_v, sem1)
        wait(buf0_v, sem0)
        pltpu.sync_copy(buf0_v, acc_sh.at[idxd_v.at[k0]], add=True)
        pltpu.async_copy(tbl_hbm.at[idxs_v.at[slc(k0 + 2)]], buf0_v, sem0)
        wait(buf1_v, sem1)
        pltpu.sync_copy(buf1_v, acc_sh.at[idxd_v.at[k1]], add=True)
        return carry

    lax.fori_loop(0, KCH // 2, pair, 0)
    wait(buf0_v, sem0)
    pltpu.sync_copy(buf0_v, acc_sh.at[idxd_v.at[KCH - 1]], add=True)
    plsc.subcore_barrier()

    def dump(j, c2):
        pltpu.sync_copy(acc_sh.at[pl.ds(sid * RPT + j * CH, CH)], buf0_v)
        pltpu.sync_copy(buf0_v,
                        out_part.at[pl.ds(cid * CB + sid * RPT + j * CH, CH)])
        return c2

    lax.fori_loop(0, RPT // CH, dump, 0)


_sc_gather_scatter = pl.kernel(
    _sc_gs_body,
    out_type=jax.ShapeDtypeStruct((CB + NP_, H), _f32),
    mesh=_mesh,
    scratch_types=[
        pltpu.VMEM((EPW,), jnp.int32),
        pltpu.VMEM((KCH, CH), jnp.int32),
        pltpu.VMEM((CH, H), _f32),
        pltpu.VMEM((CH, H), _f32),
        pltpu.VMEM_SHARED((NP_, H), _f32),
        pltpu.SemaphoreType.DMA,
        pltpu.SemaphoreType.DMA,
    ],
)


def _sc_agg_body(ean_hbm, dst2_hbm, out_part,
                 idxd_v, buf0_v, buf1_v, acc_sh, sem0, sem1):
    cid = lax.axis_index("c")
    sid = lax.axis_index("s")
    wid = sid * NC + cid
    pltpu.sync_copy(dst2_hbm.at[wid], idxd_v)

    def zrow(r, c2):
        for lane in range(H // 16):
            buf0_v[r, pl.ds(lane * 16, 16)] = jnp.zeros((16,), _f32)
        return c2

    lax.fori_loop(0, CH, zrow, 0)

    def zcopy(j, c2):
        pltpu.sync_copy(buf0_v, acc_sh.at[pl.ds(sid * RPT + j * CH, CH)])
        return c2

    lax.fori_loop(0, RPT // CH, zcopy, 0)
    plsc.subcore_barrier()

    def ld(k, buf, sem):
        pltpu.async_copy(ean_hbm.at[pl.ds(wid * EPW + k * CH, CH)], buf, sem)

    def wait(buf, sem):
        pltpu.make_async_copy(ean_hbm.at[pl.ds(0, CH)], buf, sem).wait()

    ld(0, buf0_v, sem0)

    def pair(p, carry):
        k0 = 2 * p
        k1 = k0 + 1
        ld(k1, buf1_v, sem1)
        wait(buf0_v, sem0)
        pltpu.sync_copy(buf0_v, acc_sh.at[idxd_v.at[k0]], add=True)
        ld(k0 + 2, buf0_v, sem0)
        wait(buf1_v, sem1)
        pltpu.sync_copy(buf1_v, acc_sh.at[idxd_v.at[k1]], add=True)
        return carry

    lax.fori_loop(0, KCH // 2, pair, 0)
    wait(buf0_v, sem0)
    pltpu.sync_copy(buf0_v, acc_sh.at[idxd_v.at[KCH - 1]], add=True)
    plsc.subcore_barrier()

    def dump(j, c2):
        pltpu.sync_copy(acc_sh.at[pl.ds(sid * RPT + j * CH, CH)], buf0_v)
        pltpu.sync_copy(buf0_v,
                        out_part.at[pl.ds(cid * NP_ + sid * RPT + j * CH, CH)])
        return c2

    lax.fori_loop(0, RPT // CH, dump, 0)


_sc_agg = pl.kernel(
    _sc_agg_body,
    out_type=jax.ShapeDtypeStruct((NC * NP_, H), _f32),
    mesh=_mesh,
    scratch_types=[
        pltpu.VMEM((KCH, CH), jnp.int32),
        pltpu.VMEM((CH, H), _f32),
        pltpu.VMEM((CH, H), _f32),
        pltpu.VMEM_SHARED((NP_, H), _f32),
        pltpu.SemaphoreType.DMA,
        pltpu.SemaphoreType.DMA,
    ],
)


def kernel(x, edge_attr, edge_index, W1e, b1e, W2e, b2e, W1n, b1n, W2n, b2n):
    src1 = edge_index[0].reshape(NW, EPW)
    dst1 = edge_index[1].reshape(NW, EPW)
    dst2 = edge_index[1].reshape(NW, KCH, CH)
    b1e2 = b1e.reshape(1, H)
    b2e2 = b2e.reshape(1, H)
    b1n2 = b1n.reshape(1, H)
    b2n2 = b2n.reshape(1, H)

    a_mat, b_mat, c_head = pl.pallas_call(
        _prep1_body,
        grid=(GN,),
        in_specs=[_n_spec(), _n_spec(), _full_spec((3 * H, H))],
        out_specs=[_n_spec()] * 3,
        out_shape=[jax.ShapeDtypeStruct((N, H), _f32)] * 3,
    )(x, edge_attr, W1e)

    degf = _sc_deg(dst2)
    g_mat = _sc_build_g(a_mat, b_mat, src1, dst1)

    u_mat, dinvh, v_head = pl.pallas_call(
        _prep2_body,
        grid=(GN,),
        in_specs=[_n_spec(), _n_spec(), _n_spec(), _n2_spec()],
        out_specs=[_n_spec()] * 3,
        out_shape=[jax.ShapeDtypeStruct((N, H), _f32)] * 3,
    )(g_mat, c_head, degf, degf)

    sepf = _sc_gather_scatter(u_mat, src1, dst2)

    _tail_spec = pl.BlockSpec((TILE, H), lambda i: (NT_H + i, 0))
    eo_t, ean_t = pl.pallas_call(
        _edge_tail_body,
        grid=(NT_E - NT_H,),
        in_specs=[_tail_spec, _tail_spec,
                  _full_spec((3 * H, H)), _full_spec((H, H)),
                  _full_spec((1, H)), _full_spec((1, H))],
        out_specs=[_tail_spec] * 2,
        out_shape=[jax.ShapeDtypeStruct((E, H), _f32)] * 2,
    )(edge_attr, g_mat, W1e, W2e, b1e2, b2e2)

    _hrow = pl.BlockSpec((TILE, H), lambda i: (i, 0))
    edge_out, ean = pl.pallas_call(
        _edge_head_body,
        grid=(NT_H,),
        in_specs=[_hrow, _hrow, _hrow, _hrow,
                  pl.BlockSpec((TILE, H), lambda i: (CB // TILE + i, 0)),
                  _full_spec((H, H)), _full_spec((1, H)), _full_spec((1, H)),
                  _hrow, _hrow],
        out_specs=[_hrow] * 2,
        out_shape=[jax.ShapeDtypeStruct((E, H), _f32)] * 2,
        input_output_aliases={8: 0, 9: 1},
    )(edge_attr, v_head, dinvh, sepf, sepf, W2e, b1e2, b2e2, eo_t, ean_t)

    aggf = _sc_agg(ean, dst2)

    u_n, v_n = pl.pallas_call(
        _prep_node_body,
        grid=(GN,),
        in_specs=[_n_spec(), _n_spec(), _n2_spec(), _n_spec(),
                  _full_spec((2 * H, H))],
        out_specs=[_n_spec()] * 2,
        out_shape=[jax.ShapeDtypeStruct((N, H), _f32)] * 2,
    )(x, aggf, aggf, dinvh, W1n)

    snf = _sc_gather_scatter(u_n, src1, dst2)

    x_out = pl.pallas_call(
        _final_body,
        grid=(GN,),
        in_specs=[_n_spec()] * 3
        + [_n_spec(), pl.BlockSpec((TN, H), lambda i: (CB // TN + i, 0))]
        + [_full_spec((H, H)), _full_spec((1, H)), _full_spec((1, H))],
        out_specs=_n_spec(),
        out_shape=jax.ShapeDtypeStruct((N, H), _f32),
    )(x, v_n, dinvh, snf, snf, W2n, b1n2, b2n2)

    return (x_out, edge_out)
